# Initial kernel scaffold; baseline (speedup 1.0000x reference)
import jax
import jax.numpy as jnp
from jax.experimental import pallas as pl


def kernel(energies, indices):
    raise NotImplementedError("write your pallas kernel here")


# speedup vs baseline: 6.2326x; 6.2326x over previous
import functools

import jax
import jax.numpy as jnp
from jax import lax
from jax.experimental import pallas as pl
from jax.experimental.pallas import tpu as pltpu
from jax.experimental.pallas import tpu_sc as plsc

_N = 16384
_VPAD = 128
_NC = 2
_NS = 16
_NW = _NC * _NS
_BPW = _N // _NW
_L = 16


def _sc_gather(table_pad, indices):
    mesh = plsc.VectorSubcoreMesh(core_axis_name="c", subcore_axis_name="s")

    @functools.partial(
        pl.kernel,
        mesh=mesh,
        out_type=jax.ShapeDtypeStruct((_N,), jnp.float32),
        scratch_types=[
            pltpu.VMEM((_VPAD,), jnp.float32),
            pltpu.VMEM((_BPW,), jnp.int32),
            pltpu.VMEM((_BPW,), jnp.float32),
        ],
        compiler_params=pltpu.CompilerParams(needs_layout_passes=False),
    )
    def k(tab_hbm, idx_hbm, out_hbm, tab_v, idx_v, out_v):
        wid = lax.axis_index("s") * _NC + lax.axis_index("c")
        base = wid * _BPW
        pltpu.sync_copy(tab_hbm, tab_v)
        pltpu.sync_copy(idx_hbm.at[pl.ds(base, _BPW)], idx_v)
        for j in range(_BPW // _L):
            iv = idx_v[pl.ds(j * _L, _L)]
            out_v[pl.ds(j * _L, _L)] = plsc.load_gather(tab_v, [iv])
        pltpu.sync_copy(out_v, out_hbm.at[pl.ds(base, _BPW)])

    return k(table_pad, indices)


def kernel(energies, indices):
    table_pad = jnp.zeros((_VPAD,), jnp.float32).at[:energies.shape[0]].set(
        energies.astype(jnp.float32))
    return _sc_gather(table_pad, indices.astype(jnp.int32))


# speedup vs baseline: 6.3960x; 1.0262x over previous
import functools

import jax
import jax.numpy as jnp
from jax import lax
from jax.experimental import pallas as pl
from jax.experimental.pallas import tpu as pltpu
from jax.experimental.pallas import tpu_sc as plsc

_N = 16384
_V = 100
_NC = 2
_NS = 16
_NW = _NC * _NS
_BPW = _N // _NW
_L = 16


def kernel(energies, indices):
    mesh = plsc.VectorSubcoreMesh(core_axis_name="c", subcore_axis_name="s")

    @functools.partial(
        pl.kernel,
        mesh=mesh,
        out_type=jax.ShapeDtypeStruct((_N,), jnp.float32),
        scratch_types=[
            pltpu.VMEM((_V,), jnp.float32),
            pltpu.VMEM((_BPW,), jnp.int32),
            pltpu.VMEM((_BPW,), jnp.float32),
            pltpu.SemaphoreType.DMA,
            pltpu.SemaphoreType.DMA,
        ],
        compiler_params=pltpu.CompilerParams(needs_layout_passes=False),
    )
    def k(tab_hbm, idx_hbm, out_hbm, tab_v, idx_v, out_v, sem_t, sem_i):
        wid = lax.axis_index("s") * _NC + lax.axis_index("c")
        base = wid * _BPW
        tab_cp = pltpu.async_copy(tab_hbm, tab_v, sem_t)
        idx_cp = pltpu.async_copy(idx_hbm.at[pl.ds(base, _BPW)], idx_v, sem_i)
        tab_cp.wait()
        idx_cp.wait()
        for j in range(_BPW // _L):
            iv = idx_v[pl.ds(j * _L, _L)]
            out_v[pl.ds(j * _L, _L)] = plsc.load_gather(tab_v, [iv])
        pltpu.sync_copy(out_v, out_hbm.at[pl.ds(base, _BPW)])

    return k(energies, indices)


# speedup vs baseline: 6.8785x; 1.0754x over previous
import functools

import jax
import jax.numpy as jnp
from jax import lax
from jax.experimental import pallas as pl
from jax.experimental.pallas import tpu as pltpu
from jax.experimental.pallas import tpu_sc as plsc

_N = 16384
_V = 100
_NC = 1
_NS = 16
_NW = _NC * _NS
_BPW = _N // _NW
_L = 16


def kernel(energies, indices):
    mesh = plsc.VectorSubcoreMesh(core_axis_name="c", subcore_axis_name="s",
                                  num_cores=1)

    @functools.partial(
        pl.kernel,
        mesh=mesh,
        out_type=jax.ShapeDtypeStruct((_N,), jnp.float32),
        scratch_types=[
            pltpu.VMEM((_V,), jnp.float32),
            pltpu.VMEM((_BPW,), jnp.int32),
            pltpu.VMEM((_BPW,), jnp.float32),
            pltpu.SemaphoreType.DMA,
            pltpu.SemaphoreType.DMA,
        ],
        compiler_params=pltpu.CompilerParams(needs_layout_passes=False),
    )
    def k(tab_hbm, idx_hbm, out_hbm, tab_v, idx_v, out_v, sem_t, sem_i):
        wid = lax.axis_index("s") * _NC + lax.axis_index("c")
        base = wid * _BPW
        tab_cp = pltpu.async_copy(tab_hbm, tab_v, sem_t)
        idx_cp = pltpu.async_copy(idx_hbm.at[pl.ds(base, _BPW)], idx_v, sem_i)
        tab_cp.wait()
        idx_cp.wait()
        for j in range(_BPW // _L):
            iv = idx_v[pl.ds(j * _L, _L)]
            out_v[pl.ds(j * _L, _L)] = plsc.load_gather(tab_v, [iv])
        pltpu.sync_copy(out_v, out_hbm.at[pl.ds(base, _BPW)])

    return k(energies, indices)


# speedup vs baseline: 6.9089x; 1.0044x over previous
import functools

import jax
import jax.numpy as jnp
from jax import lax
from jax.experimental import pallas as pl
from jax.experimental.pallas import tpu as pltpu
from jax.experimental.pallas import tpu_sc as plsc

_N = 16384
_V = 100
_NC = 1
_NS = 16
_NW = _NC * _NS
_BPW = _N // _NW
_L = 16


def kernel(energies, indices):
    mesh = plsc.VectorSubcoreMesh(core_axis_name="c", subcore_axis_name="s",
                                  num_cores=1)

    @functools.partial(
        pl.kernel,
        mesh=mesh,
        out_type=jax.ShapeDtypeStruct((_N,), jnp.float32),
        scratch_types=[
            pltpu.VMEM((_V,), jnp.float32),
            pltpu.VMEM((_BPW,), jnp.int32),
            pltpu.VMEM((_BPW,), jnp.float32),
            pltpu.SemaphoreType.DMA,
            pltpu.SemaphoreType.DMA,
        ],
        compiler_params=pltpu.CompilerParams(needs_layout_passes=False),
    )
    def k(tab_hbm, idx_hbm, out_hbm, tab_v, idx_v, out_v, sem_t, sem_i):
        wid = lax.axis_index("s") * _NC + lax.axis_index("c")
        base = wid * _BPW
        tab_cp = pltpu.async_copy(tab_hbm, tab_v, sem_t)
        idx_cp = pltpu.async_copy(idx_hbm.at[pl.ds(base, _BPW)], idx_v, sem_i)
        tab_cp.wait()
        idx_cp.wait()
        def body(j, carry):
            iv = idx_v[pl.ds(j * _L, _L)]
            out_v[pl.ds(j * _L, _L)] = plsc.load_gather(tab_v, [iv])
            return carry

        lax.fori_loop(0, _BPW // _L, body, 0, unroll=4)
        pltpu.sync_copy(out_v, out_hbm.at[pl.ds(base, _BPW)])

    return k(energies, indices)
